# baseline (speedup 1.0000x reference)
import functools

import jax
import jax.numpy as jnp
from jax import lax
from jax.experimental import pallas as pl
from jax.experimental.pallas import tpu as pltpu
from jax.experimental.pallas import tpu_sc as plsc

B, N, C, NPOINT, K = 16, 2048, 256, 512, 32
CNT_TOT = float(B * NPOINT * K)
EPS = 1e-5


def _lrelu(v):
    return jnp.where(v >= 0, v, 0.1 * v)


def _bn_coeffs(acc, g, bt):
    mean = acc[0:1] / CNT_TOT
    var = acc[1:2] / CNT_TOT - mean * mean
    scale = g * jax.lax.rsqrt(var + EPS)
    off = bt - mean * scale
    return scale, off


def _fps_body(x_ref, y_ref, z_ref, cent_ref, nx_ref, ny_ref, nz_ref):
    x = x_ref[...]
    y = y_ref[...]
    z = z_ref[...]
    iota = jax.lax.broadcasted_iota(jnp.int32, (B, N), 1)

    def step(i, carry):
        distance, farthest = carry
        cent_ref[pl.ds(i, 1), :] = farthest.reshape(1, B)
        mask = iota == farthest
        zero = jnp.zeros_like(x)
        cx = jnp.sum(jnp.where(mask, x, zero), axis=1, keepdims=True)
        cy = jnp.sum(jnp.where(mask, y, zero), axis=1, keepdims=True)
        cz = jnp.sum(jnp.where(mask, z, zero), axis=1, keepdims=True)
        nx_ref[pl.ds(i, 1), :] = cx.reshape(1, B)
        ny_ref[pl.ds(i, 1), :] = cy.reshape(1, B)
        nz_ref[pl.ds(i, 1), :] = cz.reshape(1, B)
        dx = x - cx
        dy = y - cy
        dz = z - cz
        dist = dx * dx + dy * dy + dz * dz
        distance = jnp.minimum(distance, dist)
        m = jnp.max(distance, axis=1, keepdims=True)
        far = jnp.min(jnp.where(distance == m, iota, N), axis=1, keepdims=True)
        return distance, far

    init = (jnp.full((B, N), 1e10, dtype=jnp.float32),
            jnp.zeros((B, 1), dtype=jnp.int32))
    jax.lax.fori_loop(0, NPOINT, step, init)


def _fps(xyz):
    xt = xyz.transpose(2, 0, 1)
    out_shapes = (
        jax.ShapeDtypeStruct((NPOINT, B), jnp.int32),
        jax.ShapeDtypeStruct((NPOINT, B), jnp.float32),
        jax.ShapeDtypeStruct((NPOINT, B), jnp.float32),
        jax.ShapeDtypeStruct((NPOINT, B), jnp.float32),
    )
    cent_t, nx, ny, nz = pl.pallas_call(_fps_body, out_shape=out_shapes)(
        xt[0], xt[1], xt[2])
    centroids = cent_t.T
    new_xyz = jnp.stack([nx.T, ny.T, nz.T], axis=-1)
    return centroids, new_xyz


_NW = 32
_HSLICE = (B * NPOINT * K) // _NW


def _hist_body(idx_hbm, out_hbm, idx_v, tab_v):
    wid = lax.axis_index("s") * 2 + lax.axis_index("c")
    base = wid * _HSLICE
    pltpu.sync_copy(idx_hbm.at[pl.ds(base, _HSLICE)], idx_v)
    zeros16 = jnp.zeros((16,), jnp.float32)
    ones16 = jnp.ones((16,), jnp.float32)

    def zbody(i, _):
        tab_v[pl.ds(i * 16, 16)] = zeros16
        return 0

    lax.fori_loop(0, N // 16, zbody, 0)

    def body(i, _):
        v = idx_v[pl.ds(i * 16, 16)]
        plsc.addupdate_scatter(tab_v, [v], ones16)
        return 0

    lax.fori_loop(0, _HSLICE // 16, body, 0)
    pltpu.sync_copy(tab_v, out_hbm.at[wid])


def _hist_sc(idx_flat):
    mesh = plsc.VectorSubcoreMesh(core_axis_name="c", subcore_axis_name="s",
                                  num_cores=2, num_subcores=16)
    fn = pl.kernel(
        _hist_body,
        out_type=jax.ShapeDtypeStruct((_NW, N), jnp.float32),
        mesh=mesh,
        scratch_types=[
            pltpu.VMEM((_HSLICE,), jnp.int32),
            pltpu.VMEM((N,), jnp.float32),
        ],
        compiler_params=pltpu.CompilerParams(needs_layout_passes=False),
    )
    part = fn(idx_flat)
    return part.reshape(B, 2, N)


_GPW = (B * NPOINT) // _NW
_GCH = 4
_NCH = _GPW // _GCH
_ROWS_CH = _GCH * K


def _gmax_body(x2_hbm, idx_hbm, out_hbm, idx_v, rows_a, rows_b, outc_v,
               sem_a, sem_b):
    wid = lax.axis_index("s") * 2 + lax.axis_index("c")
    ibase = wid * _GPW * K
    pltpu.sync_copy(idx_hbm.at[pl.ds(ibase, _GPW * K)], idx_v)
    badd = jnp.full((16,), (wid // 2) * N, dtype=jnp.int32)

    def addb(i, _):
        idx_v[pl.ds(i * 16, 16)] = idx_v[pl.ds(i * 16, 16)] + badd
        return 0

    lax.fori_loop(0, (_GPW * K) // 16, addb, 0)

    def fire(c, rows_v, sem):
        pltpu.async_copy(
            x2_hbm.at[idx_v.at[pl.ds(c * _ROWS_CH, _ROWS_CH)]], rows_v, sem)

    def wait(rows_v, sem):
        pltpu.make_async_copy(
            x2_hbm.at[idx_v.at[pl.ds(0, _ROWS_CH)]], rows_v, sem).wait()

    fire(0, rows_a, sem_a)
    fire(1, rows_b, sem_b)

    def process(c, rows_v):
        for g in range(_GCH):
            accs = [rows_v[g * K, pl.ds(h * 16, 16)] for h in range(16)]

            def rbody(r, accs):
                return tuple(
                    jnp.maximum(a, rows_v[g * K + r, pl.ds(h * 16, 16)])
                    for h, a in enumerate(accs))

            accs = lax.fori_loop(1, K, rbody, tuple(accs))
            for h in range(16):
                outc_v[g, pl.ds(h * 16, 16)] = accs[h]
        pltpu.sync_copy(outc_v,
                        out_hbm.at[pl.ds(wid * _GPW + c * _GCH, _GCH)])

    def pair(p, _):
        c0 = 2 * p
        wait(rows_a, sem_a)
        process(c0, rows_a)

        @pl.when(c0 + 2 < _NCH)
        def _():
            fire(c0 + 2, rows_a, sem_a)

        wait(rows_b, sem_b)
        process(c0 + 1, rows_b)

        @pl.when(c0 + 3 < _NCH)
        def _():
            fire(c0 + 3, rows_b, sem_b)

        return 0

    lax.fori_loop(0, _NCH // 2, pair, 0)


def _gmax_sc(x2_flat, idx_flat):
    mesh = plsc.VectorSubcoreMesh(core_axis_name="c", subcore_axis_name="s",
                                  num_cores=2, num_subcores=16)
    fn = pl.kernel(
        _gmax_body,
        out_type=jax.ShapeDtypeStruct((B * NPOINT, C), jnp.float32),
        mesh=mesh,
        scratch_types=[
            pltpu.VMEM((_GPW * K,), jnp.int32),
            pltpu.VMEM((_ROWS_CH, C), jnp.float32),
            pltpu.VMEM((_ROWS_CH, C), jnp.float32),
            pltpu.VMEM((_GCH, C), jnp.float32),
            pltpu.SemaphoreType.DMA,
            pltpu.SemaphoreType.DMA,
        ],
        compiler_params=pltpu.CompilerParams(needs_layout_passes=False),
    )
    return fn(x2_flat, idx_flat)


def _acc_update(acc_ref, cnt_ref, y):
    cnt = cnt_ref[0]
    c1 = cnt[0:1] + cnt[1:2]
    ws = jnp.dot(c1, y, preferred_element_type=jnp.float32)
    wsq = jnp.dot(c1, y * y, preferred_element_type=jnp.float32)

    @pl.when(pl.program_id(0) == 0)
    def _():
        acc_ref[...] = jnp.zeros_like(acc_ref)

    acc_ref[...] += jnp.concatenate([ws, wsq], axis=0)


def _s1_body(f_ref, cnt_ref, w_ref, b_ref, y_ref, acc_ref):
    y = jnp.dot(f_ref[0], w_ref[...], preferred_element_type=jnp.float32)
    y = y + b_ref[...]
    y_ref[0] = y
    _acc_update(acc_ref, cnt_ref, y)


def _smid_body(yp_ref, cnt_ref, st_ref, g_ref, bt_ref, w_ref, b_ref,
               y_ref, acc_ref):
    scale, off = _bn_coeffs(st_ref[...], g_ref[...], bt_ref[...])
    x = _lrelu(yp_ref[0] * scale + off)
    y = jnp.dot(x, w_ref[...], preferred_element_type=jnp.float32)
    y = y + b_ref[...]
    y_ref[0] = y
    _acc_update(acc_ref, cnt_ref, y)


def _s4_body(y3_ref, y1_ref, cnt_ref, st3_ref, st1_ref, g3_ref, bt3_ref,
             g1_ref, bt1_ref, w_ref, b_ref, x1_ref, y_ref, acc_ref):
    scale3, off3 = _bn_coeffs(st3_ref[...], g3_ref[...], bt3_ref[...])
    scale1, off1 = _bn_coeffs(st1_ref[...], g1_ref[...], bt1_ref[...])
    h2 = y3_ref[0] * scale3 + off3
    xt = _lrelu(y1_ref[0] * scale1 + off1)
    x1 = _lrelu(h2 + xt)
    x1_ref[0] = x1
    y = jnp.dot(x1, w_ref[...], preferred_element_type=jnp.float32)
    y = y + b_ref[...]
    y_ref[0] = y
    _acc_update(acc_ref, cnt_ref, y)


def _s6_body(y5_ref, x1_ref, st5_ref, g5_ref, bt5_ref, x2_ref):
    scale5, off5 = _bn_coeffs(st5_ref[...], g5_ref[...], bt5_ref[...])
    x2_ref[0] = _lrelu(y5_ref[0] * scale5 + off5 + x1_ref[0])


_ROWS = pl.BlockSpec((1, N, C), lambda b: (b, 0, 0))
_CNT = pl.BlockSpec((1, 2, N), lambda b: (b, 0, 0))
_MAT = pl.BlockSpec((C, C), lambda b: (0, 0))
_VEC = pl.BlockSpec((1, C), lambda b: (0, 0))
_ACC = pl.BlockSpec((2, C), lambda b: (0, 0))

_ROWS_SHAPE = jax.ShapeDtypeStruct((B, N, C), jnp.float32)
_ACC_SHAPE = jax.ShapeDtypeStruct((2, C), jnp.float32)


def _stage1(f, cnt2, wt, bvec):
    return pl.pallas_call(
        _s1_body,
        grid=(B,),
        in_specs=[_ROWS, _CNT, _MAT, _VEC],
        out_specs=(_ROWS, _ACC),
        out_shape=(_ROWS_SHAPE, _ACC_SHAPE),
    )(f, cnt2, wt, bvec)


def _stage_mid(yp, cnt2, st, g, bt, wt, bvec):
    return pl.pallas_call(
        _smid_body,
        grid=(B,),
        in_specs=[_ROWS, _CNT, _ACC, _VEC, _VEC, _MAT, _VEC],
        out_specs=(_ROWS, _ACC),
        out_shape=(_ROWS_SHAPE, _ACC_SHAPE),
    )(yp, cnt2, st, g, bt, wt, bvec)


def _stage4(y3, y1, cnt2, st3, st1, g3, bt3, g1, bt1, wt, bvec):
    return pl.pallas_call(
        _s4_body,
        grid=(B,),
        in_specs=[_ROWS, _ROWS, _CNT, _ACC, _ACC, _VEC, _VEC, _VEC, _VEC,
                  _MAT, _VEC],
        out_specs=(_ROWS, _ROWS, _ACC),
        out_shape=(_ROWS_SHAPE, _ROWS_SHAPE, _ACC_SHAPE),
    )(y3, y1, cnt2, st3, st1, g3, bt3, g1, bt1, wt, bvec)


def _stage6(y5, x1, st5, g5, bt5):
    return pl.pallas_call(
        _s6_body,
        grid=(B,),
        in_specs=[_ROWS, _ROWS, _ACC, _VEC, _VEC],
        out_specs=_ROWS,
        out_shape=_ROWS_SHAPE,
    )(y5, x1, st5, g5, bt5)


def kernel(xyz, features, params):
    centroids, new_xyz = _fps(xyz)

    dist = -2.0 * jnp.matmul(new_xyz, xyz.transpose(0, 2, 1))
    dist = dist + jnp.sum(new_xyz ** 2, -1)[:, :, None]
    dist = dist + jnp.sum(xyz ** 2, -1)[:, None, :]
    idx = jnp.argsort(dist, axis=-1)[:, :, :K]

    cnt2 = _hist_sc(idx.reshape(-1).astype(jnp.int32))

    p = params
    v = lambda nm: p[nm].reshape(1, C)
    wT = lambda nm: p[nm].T

    y1, a1 = _stage1(features, cnt2, wT('W_t'), v('b_t'))
    y2, a2 = _stage_mid(y1, cnt2, a1, v('g_t'), v('bt_t'),
                        wT('W_r1a'), v('b_r1a'))
    y3, a3 = _stage_mid(y2, cnt2, a2, v('g_r1a'), v('bt_r1a'),
                        wT('W_r1b'), v('b_r1b'))
    x1, y4, a4 = _stage4(y3, y1, cnt2, a3, a1, v('g_r1b'), v('bt_r1b'),
                         v('g_t'), v('bt_t'), wT('W_r2a'), v('b_r2a'))
    y5, a5 = _stage_mid(y4, cnt2, a4, v('g_r2a'), v('bt_r2a'),
                        wT('W_r2b'), v('b_r2b'))
    x2 = _stage6(y5, x1, a5, v('g_r2b'), v('bt_r2b'))

    gm = _gmax_sc(x2.reshape(B * N, C), idx.reshape(-1).astype(jnp.int32))
    sub_features = gm.reshape(B, NPOINT, C).transpose(0, 2, 1)

    return (new_xyz.transpose(0, 2, 1), sub_features)


# speedup vs baseline: 3.2865x; 3.2865x over previous
import functools

import jax
import jax.numpy as jnp
from jax import lax
from jax.experimental import pallas as pl
from jax.experimental.pallas import tpu as pltpu
from jax.experimental.pallas import tpu_sc as plsc

B, N, C, NPOINT, K = 16, 2048, 256, 512, 32
CNT_TOT = float(B * NPOINT * K)
EPS = 1e-5


def _lrelu(v):
    return jnp.where(v >= 0, v, 0.1 * v)


def _bn_coeffs(acc, g, bt):
    mean = acc[0:1] / CNT_TOT
    var = acc[1:2] / CNT_TOT - mean * mean
    scale = g * jax.lax.rsqrt(var + EPS)
    off = bt - mean * scale
    return scale, off


def _fps_body(x_ref, y_ref, z_ref, cent_ref, nx_ref, ny_ref, nz_ref):
    x = x_ref[...]
    y = y_ref[...]
    z = z_ref[...]
    iota = jax.lax.broadcasted_iota(jnp.int32, (B, N), 1)

    def step(i, carry):
        distance, farthest = carry
        cent_ref[pl.ds(i, 1), :] = farthest.reshape(1, B)
        mask = iota == farthest
        zero = jnp.zeros_like(x)
        cx = jnp.sum(jnp.where(mask, x, zero), axis=1, keepdims=True)
        cy = jnp.sum(jnp.where(mask, y, zero), axis=1, keepdims=True)
        cz = jnp.sum(jnp.where(mask, z, zero), axis=1, keepdims=True)
        nx_ref[pl.ds(i, 1), :] = cx.reshape(1, B)
        ny_ref[pl.ds(i, 1), :] = cy.reshape(1, B)
        nz_ref[pl.ds(i, 1), :] = cz.reshape(1, B)
        dx = x - cx
        dy = y - cy
        dz = z - cz
        dist = dx * dx + dy * dy + dz * dz
        distance = jnp.minimum(distance, dist)
        m = jnp.max(distance, axis=1, keepdims=True)
        far = jnp.min(jnp.where(distance == m, iota, N), axis=1, keepdims=True)
        return distance, far

    init = (jnp.full((B, N), 1e10, dtype=jnp.float32),
            jnp.zeros((B, 1), dtype=jnp.int32))
    jax.lax.fori_loop(0, NPOINT, step, init)


def _fps(xyz):
    xt = xyz.transpose(2, 0, 1)
    out_shapes = (
        jax.ShapeDtypeStruct((NPOINT, B), jnp.int32),
        jax.ShapeDtypeStruct((NPOINT, B), jnp.float32),
        jax.ShapeDtypeStruct((NPOINT, B), jnp.float32),
        jax.ShapeDtypeStruct((NPOINT, B), jnp.float32),
    )
    cent_t, nx, ny, nz = pl.pallas_call(_fps_body, out_shape=out_shapes)(
        xt[0], xt[1], xt[2])
    centroids = cent_t.T
    new_xyz = jnp.stack([nx.T, ny.T, nz.T], axis=-1)
    return centroids, new_xyz


_NW = 32
_HSLICE = (B * NPOINT * K) // _NW


def _hist_body(idx_hbm, out_hbm, idx_v, tab_v):
    wid = lax.axis_index("s") * 2 + lax.axis_index("c")
    base = wid * _HSLICE
    pltpu.sync_copy(idx_hbm.at[pl.ds(base, _HSLICE)], idx_v)
    zeros16 = jnp.zeros((16,), jnp.float32)
    ones16 = jnp.ones((16,), jnp.float32)

    def zbody(i, _):
        tab_v[pl.ds(i * 16, 16)] = zeros16
        return 0

    lax.fori_loop(0, N // 16, zbody, 0)

    def body(i, _):
        v = idx_v[pl.ds(i * 16, 16)]
        plsc.addupdate_scatter(tab_v, [v], ones16)
        return 0

    lax.fori_loop(0, _HSLICE // 16, body, 0)
    pltpu.sync_copy(tab_v, out_hbm.at[wid])


def _hist_sc(idx_flat):
    mesh = plsc.VectorSubcoreMesh(core_axis_name="c", subcore_axis_name="s",
                                  num_cores=2, num_subcores=16)
    fn = pl.kernel(
        _hist_body,
        out_type=jax.ShapeDtypeStruct((_NW, N), jnp.float32),
        mesh=mesh,
        scratch_types=[
            pltpu.VMEM((_HSLICE,), jnp.int32),
            pltpu.VMEM((N,), jnp.float32),
        ],
        compiler_params=pltpu.CompilerParams(needs_layout_passes=False),
    )
    part = fn(idx_flat)
    return part.reshape(B, 2, N)


_NBIG = N
_FINF = 3.4e38


def _topk_body(dist_ref, idx_ref, d_ref):
    d_ref[...] = dist_ref[0]
    iota = jax.lax.broadcasted_iota(jnp.int32, (N, NPOINT), 0)

    def step(j, _):
        d = d_ref[...]
        m = jnp.min(d, axis=0, keepdims=True)
        cand = jnp.where(d == m, iota, _NBIG)
        sel = jnp.min(cand, axis=0, keepdims=True)
        idx_ref[0, pl.ds(j, 1), :] = sel
        d_ref[...] = jnp.where(iota == sel, _FINF, d)
        return 0

    jax.lax.fori_loop(0, K, step, 0, unroll=2)


def _topk(xyz, new_xyz_t):
    new_xyz = new_xyz_t.transpose(0, 2, 1)
    dist = -2.0 * jnp.matmul(new_xyz, xyz.transpose(0, 2, 1))
    dist = dist + jnp.sum(new_xyz ** 2, -1)[:, :, None]
    dist = dist + jnp.sum(xyz ** 2, -1)[:, None, :]
    dist_t = dist.transpose(0, 2, 1)
    idx_t = pl.pallas_call(
        _topk_body,
        grid=(B,),
        in_specs=[pl.BlockSpec((1, N, NPOINT), lambda b: (b, 0, 0))],
        out_specs=pl.BlockSpec((1, K, NPOINT), lambda b: (b, 0, 0)),
        out_shape=jax.ShapeDtypeStruct((B, K, NPOINT), jnp.int32),
        scratch_shapes=[pltpu.VMEM((N, NPOINT), jnp.float32)],
    )(dist_t)
    return idx_t.transpose(0, 2, 1)


_GPW = (B * NPOINT) // _NW
_GCH = 4
_NCH = _GPW // _GCH
_ROWS_CH = _GCH * K


def _gmax_body(x2_hbm, idx_hbm, out_hbm, idx_v, rows_a, rows_b, outc_v,
               sem_a, sem_b):
    wid = lax.axis_index("s") * 2 + lax.axis_index("c")
    ibase = wid * _GPW * K
    pltpu.sync_copy(idx_hbm.at[pl.ds(ibase, _GPW * K)], idx_v)
    badd = jnp.full((16,), (wid // 2) * N, dtype=jnp.int32)

    def addb(i, _):
        idx_v[pl.ds(i * 16, 16)] = idx_v[pl.ds(i * 16, 16)] + badd
        return 0

    lax.fori_loop(0, (_GPW * K) // 16, addb, 0)

    def fire(c, rows_v, sem):
        pltpu.async_copy(
            x2_hbm.at[idx_v.at[pl.ds(c * _ROWS_CH, _ROWS_CH)]], rows_v, sem)

    def wait(rows_v, sem):
        pltpu.make_async_copy(
            x2_hbm.at[idx_v.at[pl.ds(0, _ROWS_CH)]], rows_v, sem).wait()

    fire(0, rows_a, sem_a)
    fire(1, rows_b, sem_b)

    def process(c, rows_v):
        for g in range(_GCH):
            accs = [rows_v[g * K, pl.ds(h * 16, 16)] for h in range(16)]

            def rbody(r, accs):
                return tuple(
                    jnp.maximum(a, rows_v[g * K + r, pl.ds(h * 16, 16)])
                    for h, a in enumerate(accs))

            accs = lax.fori_loop(1, K, rbody, tuple(accs))
            for h in range(16):
                outc_v[g, pl.ds(h * 16, 16)] = accs[h]
        pltpu.sync_copy(outc_v,
                        out_hbm.at[pl.ds(wid * _GPW + c * _GCH, _GCH)])

    def pair(p, _):
        c0 = 2 * p
        wait(rows_a, sem_a)
        process(c0, rows_a)

        @pl.when(c0 + 2 < _NCH)
        def _():
            fire(c0 + 2, rows_a, sem_a)

        wait(rows_b, sem_b)
        process(c0 + 1, rows_b)

        @pl.when(c0 + 3 < _NCH)
        def _():
            fire(c0 + 3, rows_b, sem_b)

        return 0

    lax.fori_loop(0, _NCH // 2, pair, 0)


def _gmax_sc(x2_flat, idx_flat):
    mesh = plsc.VectorSubcoreMesh(core_axis_name="c", subcore_axis_name="s",
                                  num_cores=2, num_subcores=16)
    fn = pl.kernel(
        _gmax_body,
        out_type=jax.ShapeDtypeStruct((B * NPOINT, C), jnp.float32),
        mesh=mesh,
        scratch_types=[
            pltpu.VMEM((_GPW * K,), jnp.int32),
            pltpu.VMEM((_ROWS_CH, C), jnp.float32),
            pltpu.VMEM((_ROWS_CH, C), jnp.float32),
            pltpu.VMEM((_GCH, C), jnp.float32),
            pltpu.SemaphoreType.DMA,
            pltpu.SemaphoreType.DMA,
        ],
        compiler_params=pltpu.CompilerParams(needs_layout_passes=False),
    )
    return fn(x2_flat, idx_flat)


def _acc_update(acc_ref, cnt_ref, y):
    cnt = cnt_ref[0]
    c1 = cnt[0:1] + cnt[1:2]
    ws = jnp.dot(c1, y, preferred_element_type=jnp.float32)
    wsq = jnp.dot(c1, y * y, preferred_element_type=jnp.float32)

    @pl.when(pl.program_id(0) == 0)
    def _():
        acc_ref[...] = jnp.zeros_like(acc_ref)

    acc_ref[...] += jnp.concatenate([ws, wsq], axis=0)


def _s1_body(f_ref, cnt_ref, w_ref, b_ref, y_ref, acc_ref):
    y = jnp.dot(f_ref[0], w_ref[...], preferred_element_type=jnp.float32)
    y = y + b_ref[...]
    y_ref[0] = y
    _acc_update(acc_ref, cnt_ref, y)


def _smid_body(yp_ref, cnt_ref, st_ref, g_ref, bt_ref, w_ref, b_ref,
               y_ref, acc_ref):
    scale, off = _bn_coeffs(st_ref[...], g_ref[...], bt_ref[...])
    x = _lrelu(yp_ref[0] * scale + off)
    y = jnp.dot(x, w_ref[...], preferred_element_type=jnp.float32)
    y = y + b_ref[...]
    y_ref[0] = y
    _acc_update(acc_ref, cnt_ref, y)


def _s4_body(y3_ref, y1_ref, cnt_ref, st3_ref, st1_ref, g3_ref, bt3_ref,
             g1_ref, bt1_ref, w_ref, b_ref, x1_ref, y_ref, acc_ref):
    scale3, off3 = _bn_coeffs(st3_ref[...], g3_ref[...], bt3_ref[...])
    scale1, off1 = _bn_coeffs(st1_ref[...], g1_ref[...], bt1_ref[...])
    h2 = y3_ref[0] * scale3 + off3
    xt = _lrelu(y1_ref[0] * scale1 + off1)
    x1 = _lrelu(h2 + xt)
    x1_ref[0] = x1
    y = jnp.dot(x1, w_ref[...], preferred_element_type=jnp.float32)
    y = y + b_ref[...]
    y_ref[0] = y
    _acc_update(acc_ref, cnt_ref, y)


def _s6_body(y5_ref, x1_ref, st5_ref, g5_ref, bt5_ref, x2_ref):
    scale5, off5 = _bn_coeffs(st5_ref[...], g5_ref[...], bt5_ref[...])
    x2_ref[0] = _lrelu(y5_ref[0] * scale5 + off5 + x1_ref[0])


_ROWS = pl.BlockSpec((1, N, C), lambda b: (b, 0, 0))
_CNT = pl.BlockSpec((1, 2, N), lambda b: (b, 0, 0))
_MAT = pl.BlockSpec((C, C), lambda b: (0, 0))
_VEC = pl.BlockSpec((1, C), lambda b: (0, 0))
_ACC = pl.BlockSpec((2, C), lambda b: (0, 0))

_ROWS_SHAPE = jax.ShapeDtypeStruct((B, N, C), jnp.float32)
_ACC_SHAPE = jax.ShapeDtypeStruct((2, C), jnp.float32)


def _stage1(f, cnt2, wt, bvec):
    return pl.pallas_call(
        _s1_body,
        grid=(B,),
        in_specs=[_ROWS, _CNT, _MAT, _VEC],
        out_specs=(_ROWS, _ACC),
        out_shape=(_ROWS_SHAPE, _ACC_SHAPE),
    )(f, cnt2, wt, bvec)


def _stage_mid(yp, cnt2, st, g, bt, wt, bvec):
    return pl.pallas_call(
        _smid_body,
        grid=(B,),
        in_specs=[_ROWS, _CNT, _ACC, _VEC, _VEC, _MAT, _VEC],
        out_specs=(_ROWS, _ACC),
        out_shape=(_ROWS_SHAPE, _ACC_SHAPE),
    )(yp, cnt2, st, g, bt, wt, bvec)


def _stage4(y3, y1, cnt2, st3, st1, g3, bt3, g1, bt1, wt, bvec):
    return pl.pallas_call(
        _s4_body,
        grid=(B,),
        in_specs=[_ROWS, _ROWS, _CNT, _ACC, _ACC, _VEC, _VEC, _VEC, _VEC,
                  _MAT, _VEC],
        out_specs=(_ROWS, _ROWS, _ACC),
        out_shape=(_ROWS_SHAPE, _ROWS_SHAPE, _ACC_SHAPE),
    )(y3, y1, cnt2, st3, st1, g3, bt3, g1, bt1, wt, bvec)


def _stage6(y5, x1, st5, g5, bt5):
    return pl.pallas_call(
        _s6_body,
        grid=(B,),
        in_specs=[_ROWS, _ROWS, _ACC, _VEC, _VEC],
        out_specs=_ROWS,
        out_shape=_ROWS_SHAPE,
    )(y5, x1, st5, g5, bt5)


def kernel(xyz, features, params):
    centroids, new_xyz = _fps(xyz)

    new_xyz_t = new_xyz.transpose(0, 2, 1)
    idx = _topk(xyz, new_xyz_t)

    cnt2 = _hist_sc(idx.reshape(-1).astype(jnp.int32))

    p = params
    v = lambda nm: p[nm].reshape(1, C)
    wT = lambda nm: p[nm].T

    y1, a1 = _stage1(features, cnt2, wT('W_t'), v('b_t'))
    y2, a2 = _stage_mid(y1, cnt2, a1, v('g_t'), v('bt_t'),
                        wT('W_r1a'), v('b_r1a'))
    y3, a3 = _stage_mid(y2, cnt2, a2, v('g_r1a'), v('bt_r1a'),
                        wT('W_r1b'), v('b_r1b'))
    x1, y4, a4 = _stage4(y3, y1, cnt2, a3, a1, v('g_r1b'), v('bt_r1b'),
                         v('g_t'), v('bt_t'), wT('W_r2a'), v('b_r2a'))
    y5, a5 = _stage_mid(y4, cnt2, a4, v('g_r2a'), v('bt_r2a'),
                        wT('W_r2b'), v('b_r2b'))
    x2 = _stage6(y5, x1, a5, v('g_r2b'), v('bt_r2b'))

    gm = _gmax_sc(x2.reshape(B * N, C), idx.reshape(-1).astype(jnp.int32))
    sub_features = gm.reshape(B, NPOINT, C).transpose(0, 2, 1)

    return (new_xyz_t, sub_features)


# speedup vs baseline: 3.2962x; 1.0030x over previous
import functools

import jax
import jax.numpy as jnp
from jax import lax
from jax.experimental import pallas as pl
from jax.experimental.pallas import tpu as pltpu
from jax.experimental.pallas import tpu_sc as plsc

B, N, C, NPOINT, K = 16, 2048, 256, 512, 32
CNT_TOT = float(B * NPOINT * K)
EPS = 1e-5


def _lrelu(v):
    return jnp.where(v >= 0, v, 0.1 * v)


def _bn_coeffs(acc, g, bt):
    mean = acc[0:1] / CNT_TOT
    var = acc[1:2] / CNT_TOT - mean * mean
    scale = g * jax.lax.rsqrt(var + EPS)
    off = bt - mean * scale
    return scale, off


def _fps_body(x_ref, y_ref, z_ref, cent_ref, nx_ref, ny_ref, nz_ref):
    x = x_ref[...]
    y = y_ref[...]
    z = z_ref[...]
    iota = jax.lax.broadcasted_iota(jnp.int32, (B, N), 1)

    def step(i, carry):
        distance, farthest = carry
        cent_ref[pl.ds(i, 1), :] = farthest.reshape(1, B)
        mask = iota == farthest
        zero = jnp.zeros_like(x)
        cx = jnp.sum(jnp.where(mask, x, zero), axis=1, keepdims=True)
        cy = jnp.sum(jnp.where(mask, y, zero), axis=1, keepdims=True)
        cz = jnp.sum(jnp.where(mask, z, zero), axis=1, keepdims=True)
        nx_ref[pl.ds(i, 1), :] = cx.reshape(1, B)
        ny_ref[pl.ds(i, 1), :] = cy.reshape(1, B)
        nz_ref[pl.ds(i, 1), :] = cz.reshape(1, B)
        dx = x - cx
        dy = y - cy
        dz = z - cz
        dist = dx * dx + dy * dy + dz * dz
        distance = jnp.minimum(distance, dist)
        m = jnp.max(distance, axis=1, keepdims=True)
        far = jnp.min(jnp.where(distance == m, iota, N), axis=1, keepdims=True)
        return distance, far

    init = (jnp.full((B, N), 1e10, dtype=jnp.float32),
            jnp.zeros((B, 1), dtype=jnp.int32))
    jax.lax.fori_loop(0, NPOINT, step, init)


def _fps(xyz):
    xt = xyz.transpose(2, 0, 1)
    out_shapes = (
        jax.ShapeDtypeStruct((NPOINT, B), jnp.int32),
        jax.ShapeDtypeStruct((NPOINT, B), jnp.float32),
        jax.ShapeDtypeStruct((NPOINT, B), jnp.float32),
        jax.ShapeDtypeStruct((NPOINT, B), jnp.float32),
    )
    cent_t, nx, ny, nz = pl.pallas_call(_fps_body, out_shape=out_shapes)(
        xt[0], xt[1], xt[2])
    centroids = cent_t.T
    new_xyz = jnp.stack([nx.T, ny.T, nz.T], axis=-1)
    return centroids, new_xyz


_NW = 32
_HSLICE = (B * NPOINT * K) // _NW


def _hist_body(idx_hbm, out_hbm, idx_v, tab_v):
    wid = lax.axis_index("s") * 2 + lax.axis_index("c")
    base = wid * _HSLICE
    pltpu.sync_copy(idx_hbm.at[pl.ds(base, _HSLICE)], idx_v)
    zeros16 = jnp.zeros((16,), jnp.float32)
    ones16 = jnp.ones((16,), jnp.float32)

    def zbody(i, _):
        tab_v[pl.ds(i * 16, 16)] = zeros16
        return 0

    lax.fori_loop(0, N // 16, zbody, 0)

    def body(i, _):
        v = idx_v[pl.ds(i * 16, 16)]
        plsc.addupdate_scatter(tab_v, [v], ones16)
        return 0

    lax.fori_loop(0, _HSLICE // 16, body, 0)
    pltpu.sync_copy(tab_v, out_hbm.at[wid])


def _hist_sc(idx_flat):
    mesh = plsc.VectorSubcoreMesh(core_axis_name="c", subcore_axis_name="s",
                                  num_cores=2, num_subcores=16)
    fn = pl.kernel(
        _hist_body,
        out_type=jax.ShapeDtypeStruct((_NW, N), jnp.float32),
        mesh=mesh,
        scratch_types=[
            pltpu.VMEM((_HSLICE,), jnp.int32),
            pltpu.VMEM((N,), jnp.float32),
        ],
        compiler_params=pltpu.CompilerParams(needs_layout_passes=False),
    )
    part = fn(idx_flat)
    return part.reshape(B, 2, N)


_NBIG = N
_FINF = 3.4e38


def _topk_body(p_ref, q_ref, xx_ref, qq_ref, idx_ref, d_ref):
    mm = jax.lax.dot_general(p_ref[0], q_ref[0], (((1,), (0,)), ((), ())),
                             preferred_element_type=jnp.float32)
    d_ref[...] = (-2.0 * mm + qq_ref[0]) + xx_ref[0]
    iota = jax.lax.broadcasted_iota(jnp.int32, (N, NPOINT), 0)

    def step(j, _):
        d = d_ref[...]
        m = jnp.min(d, axis=0, keepdims=True)
        cand = jnp.where(d == m, iota, _NBIG)
        sel = jnp.min(cand, axis=0, keepdims=True)
        idx_ref[0, pl.ds(j, 1), :] = sel
        d_ref[...] = jnp.where(iota == sel, _FINF, d)
        return 0

    jax.lax.fori_loop(0, K, step, 0, unroll=2)


def _topk(xyz, new_xyz_t):
    xx = jnp.sum(xyz ** 2, -1)[:, :, None]
    qq = jnp.sum(new_xyz_t.transpose(0, 2, 1) ** 2, -1)[:, None, :]
    idx_t = pl.pallas_call(
        _topk_body,
        grid=(B,),
        in_specs=[pl.BlockSpec((1, N, 3), lambda b: (b, 0, 0)),
                  pl.BlockSpec((1, 3, NPOINT), lambda b: (b, 0, 0)),
                  pl.BlockSpec((1, N, 1), lambda b: (b, 0, 0)),
                  pl.BlockSpec((1, 1, NPOINT), lambda b: (b, 0, 0))],
        out_specs=pl.BlockSpec((1, K, NPOINT), lambda b: (b, 0, 0)),
        out_shape=jax.ShapeDtypeStruct((B, K, NPOINT), jnp.int32),
        scratch_shapes=[pltpu.VMEM((N, NPOINT), jnp.float32)],
    )(xyz, new_xyz_t, xx, qq)
    return idx_t.transpose(0, 2, 1)


_GPW = (B * NPOINT) // _NW
_GCH = 4
_NCH = _GPW // _GCH
_ROWS_CH = _GCH * K


def _gmax_body(x2_hbm, idx_hbm, out_hbm, idx_v, rows_a, rows_b, outc_v,
               sem_a, sem_b):
    wid = lax.axis_index("s") * 2 + lax.axis_index("c")
    ibase = wid * _GPW * K
    pltpu.sync_copy(idx_hbm.at[pl.ds(ibase, _GPW * K)], idx_v)
    badd = jnp.full((16,), (wid // 2) * N, dtype=jnp.int32)

    def addb(i, _):
        idx_v[pl.ds(i * 16, 16)] = idx_v[pl.ds(i * 16, 16)] + badd
        return 0

    lax.fori_loop(0, (_GPW * K) // 16, addb, 0)

    def fire(c, rows_v, sem):
        pltpu.async_copy(
            x2_hbm.at[idx_v.at[pl.ds(c * _ROWS_CH, _ROWS_CH)]], rows_v, sem)

    def wait(rows_v, sem):
        pltpu.make_async_copy(
            x2_hbm.at[idx_v.at[pl.ds(0, _ROWS_CH)]], rows_v, sem).wait()

    fire(0, rows_a, sem_a)
    fire(1, rows_b, sem_b)

    def process(c, rows_v):
        for g in range(_GCH):
            accs = [rows_v[g * K, pl.ds(h * 16, 16)] for h in range(16)]

            def rbody(r, accs):
                return tuple(
                    jnp.maximum(a, rows_v[g * K + r, pl.ds(h * 16, 16)])
                    for h, a in enumerate(accs))

            accs = lax.fori_loop(1, K, rbody, tuple(accs))
            for h in range(16):
                outc_v[g, pl.ds(h * 16, 16)] = accs[h]
        pltpu.sync_copy(outc_v,
                        out_hbm.at[pl.ds(wid * _GPW + c * _GCH, _GCH)])

    def pair(p, _):
        c0 = 2 * p
        wait(rows_a, sem_a)
        process(c0, rows_a)

        @pl.when(c0 + 2 < _NCH)
        def _():
            fire(c0 + 2, rows_a, sem_a)

        wait(rows_b, sem_b)
        process(c0 + 1, rows_b)

        @pl.when(c0 + 3 < _NCH)
        def _():
            fire(c0 + 3, rows_b, sem_b)

        return 0

    lax.fori_loop(0, _NCH // 2, pair, 0)


def _gmax_sc(x2_flat, idx_flat):
    mesh = plsc.VectorSubcoreMesh(core_axis_name="c", subcore_axis_name="s",
                                  num_cores=2, num_subcores=16)
    fn = pl.kernel(
        _gmax_body,
        out_type=jax.ShapeDtypeStruct((B * NPOINT, C), jnp.float32),
        mesh=mesh,
        scratch_types=[
            pltpu.VMEM((_GPW * K,), jnp.int32),
            pltpu.VMEM((_ROWS_CH, C), jnp.float32),
            pltpu.VMEM((_ROWS_CH, C), jnp.float32),
            pltpu.VMEM((_GCH, C), jnp.float32),
            pltpu.SemaphoreType.DMA,
            pltpu.SemaphoreType.DMA,
        ],
        compiler_params=pltpu.CompilerParams(needs_layout_passes=False),
    )
    return fn(x2_flat, idx_flat)


def _acc_update(acc_ref, cnt_ref, y):
    cnt = cnt_ref[0]
    c1 = cnt[0:1] + cnt[1:2]
    ws = jnp.dot(c1, y, preferred_element_type=jnp.float32)
    wsq = jnp.dot(c1, y * y, preferred_element_type=jnp.float32)

    @pl.when(pl.program_id(0) == 0)
    def _():
        acc_ref[...] = jnp.zeros_like(acc_ref)

    acc_ref[...] += jnp.concatenate([ws, wsq], axis=0)


def _s1_body(f_ref, cnt_ref, w_ref, b_ref, y_ref, acc_ref):
    y = jnp.dot(f_ref[0], w_ref[...], preferred_element_type=jnp.float32)
    y = y + b_ref[...]
    y_ref[0] = y
    _acc_update(acc_ref, cnt_ref, y)


def _smid_body(yp_ref, cnt_ref, st_ref, g_ref, bt_ref, w_ref, b_ref,
               y_ref, acc_ref):
    scale, off = _bn_coeffs(st_ref[...], g_ref[...], bt_ref[...])
    x = _lrelu(yp_ref[0] * scale + off)
    y = jnp.dot(x, w_ref[...], preferred_element_type=jnp.float32)
    y = y + b_ref[...]
    y_ref[0] = y
    _acc_update(acc_ref, cnt_ref, y)


def _s4_body(y3_ref, y1_ref, cnt_ref, st3_ref, st1_ref, g3_ref, bt3_ref,
             g1_ref, bt1_ref, w_ref, b_ref, x1_ref, y_ref, acc_ref):
    scale3, off3 = _bn_coeffs(st3_ref[...], g3_ref[...], bt3_ref[...])
    scale1, off1 = _bn_coeffs(st1_ref[...], g1_ref[...], bt1_ref[...])
    h2 = y3_ref[0] * scale3 + off3
    xt = _lrelu(y1_ref[0] * scale1 + off1)
    x1 = _lrelu(h2 + xt)
    x1_ref[0] = x1
    y = jnp.dot(x1, w_ref[...], preferred_element_type=jnp.float32)
    y = y + b_ref[...]
    y_ref[0] = y
    _acc_update(acc_ref, cnt_ref, y)


def _s6_body(y5_ref, x1_ref, st5_ref, g5_ref, bt5_ref, x2_ref):
    scale5, off5 = _bn_coeffs(st5_ref[...], g5_ref[...], bt5_ref[...])
    x2_ref[0] = _lrelu(y5_ref[0] * scale5 + off5 + x1_ref[0])


_ROWS = pl.BlockSpec((1, N, C), lambda b: (b, 0, 0))
_CNT = pl.BlockSpec((1, 2, N), lambda b: (b, 0, 0))
_MAT = pl.BlockSpec((C, C), lambda b: (0, 0))
_VEC = pl.BlockSpec((1, C), lambda b: (0, 0))
_ACC = pl.BlockSpec((2, C), lambda b: (0, 0))

_ROWS_SHAPE = jax.ShapeDtypeStruct((B, N, C), jnp.float32)
_ACC_SHAPE = jax.ShapeDtypeStruct((2, C), jnp.float32)


def _stage1(f, cnt2, wt, bvec):
    return pl.pallas_call(
        _s1_body,
        grid=(B,),
        in_specs=[_ROWS, _CNT, _MAT, _VEC],
        out_specs=(_ROWS, _ACC),
        out_shape=(_ROWS_SHAPE, _ACC_SHAPE),
    )(f, cnt2, wt, bvec)


def _stage_mid(yp, cnt2, st, g, bt, wt, bvec):
    return pl.pallas_call(
        _smid_body,
        grid=(B,),
        in_specs=[_ROWS, _CNT, _ACC, _VEC, _VEC, _MAT, _VEC],
        out_specs=(_ROWS, _ACC),
        out_shape=(_ROWS_SHAPE, _ACC_SHAPE),
    )(yp, cnt2, st, g, bt, wt, bvec)


def _stage4(y3, y1, cnt2, st3, st1, g3, bt3, g1, bt1, wt, bvec):
    return pl.pallas_call(
        _s4_body,
        grid=(B,),
        in_specs=[_ROWS, _ROWS, _CNT, _ACC, _ACC, _VEC, _VEC, _VEC, _VEC,
                  _MAT, _VEC],
        out_specs=(_ROWS, _ROWS, _ACC),
        out_shape=(_ROWS_SHAPE, _ROWS_SHAPE, _ACC_SHAPE),
    )(y3, y1, cnt2, st3, st1, g3, bt3, g1, bt1, wt, bvec)


def _stage6(y5, x1, st5, g5, bt5):
    return pl.pallas_call(
        _s6_body,
        grid=(B,),
        in_specs=[_ROWS, _ROWS, _ACC, _VEC, _VEC],
        out_specs=_ROWS,
        out_shape=_ROWS_SHAPE,
    )(y5, x1, st5, g5, bt5)


def kernel(xyz, features, params):
    centroids, new_xyz = _fps(xyz)

    new_xyz_t = new_xyz.transpose(0, 2, 1)
    idx = _topk(xyz, new_xyz_t)

    cnt2 = _hist_sc(idx.reshape(-1).astype(jnp.int32))

    p = params
    v = lambda nm: p[nm].reshape(1, C)
    wT = lambda nm: p[nm].T

    y1, a1 = _stage1(features, cnt2, wT('W_t'), v('b_t'))
    y2, a2 = _stage_mid(y1, cnt2, a1, v('g_t'), v('bt_t'),
                        wT('W_r1a'), v('b_r1a'))
    y3, a3 = _stage_mid(y2, cnt2, a2, v('g_r1a'), v('bt_r1a'),
                        wT('W_r1b'), v('b_r1b'))
    x1, y4, a4 = _stage4(y3, y1, cnt2, a3, a1, v('g_r1b'), v('bt_r1b'),
                         v('g_t'), v('bt_t'), wT('W_r2a'), v('b_r2a'))
    y5, a5 = _stage_mid(y4, cnt2, a4, v('g_r2a'), v('bt_r2a'),
                        wT('W_r2b'), v('b_r2b'))
    x2 = _stage6(y5, x1, a5, v('g_r2b'), v('bt_r2b'))

    gm = _gmax_sc(x2.reshape(B * N, C), idx.reshape(-1).astype(jnp.int32))
    sub_features = gm.reshape(B, NPOINT, C).transpose(0, 2, 1)

    return (new_xyz_t, sub_features)


# speedup vs baseline: 3.7202x; 1.1286x over previous
import functools

import jax
import jax.numpy as jnp
from jax import lax
from jax.experimental import pallas as pl
from jax.experimental.pallas import tpu as pltpu
from jax.experimental.pallas import tpu_sc as plsc

B, N, C, NPOINT, K = 16, 2048, 256, 512, 32
CNT_TOT = float(B * NPOINT * K)
EPS = 1e-5


def _lrelu(v):
    return jnp.where(v >= 0, v, 0.1 * v)


def _bn_coeffs(acc, g, bt):
    mean = acc[0:1] / CNT_TOT
    var = acc[1:2] / CNT_TOT - mean * mean
    scale = g * jax.lax.rsqrt(var + EPS)
    off = bt - mean * scale
    return scale, off


def _fps_body(x_ref, y_ref, z_ref, cent_ref, nx_ref, ny_ref, nz_ref):
    x = x_ref[...]
    y = y_ref[...]
    z = z_ref[...]
    iota = jax.lax.broadcasted_iota(jnp.int32, (B, N), 1)

    def step(i, carry):
        distance, farthest = carry
        cent_ref[pl.ds(i, 1), :] = farthest.reshape(1, B)
        mask = iota == farthest
        zero = jnp.zeros_like(x)
        cx = jnp.sum(jnp.where(mask, x, zero), axis=1, keepdims=True)
        cy = jnp.sum(jnp.where(mask, y, zero), axis=1, keepdims=True)
        cz = jnp.sum(jnp.where(mask, z, zero), axis=1, keepdims=True)
        nx_ref[pl.ds(i, 1), :] = cx.reshape(1, B)
        ny_ref[pl.ds(i, 1), :] = cy.reshape(1, B)
        nz_ref[pl.ds(i, 1), :] = cz.reshape(1, B)
        dx = x - cx
        dy = y - cy
        dz = z - cz
        dist = dx * dx + dy * dy + dz * dz
        distance = jnp.minimum(distance, dist)
        m = jnp.max(distance, axis=1, keepdims=True)
        far = jnp.min(jnp.where(distance == m, iota, N), axis=1, keepdims=True)
        return distance, far

    init = (jnp.full((B, N), 1e10, dtype=jnp.float32),
            jnp.zeros((B, 1), dtype=jnp.int32))
    jax.lax.fori_loop(0, NPOINT, step, init)


def _fps(xyz):
    xt = xyz.transpose(2, 0, 1)
    out_shapes = (
        jax.ShapeDtypeStruct((NPOINT, B), jnp.int32),
        jax.ShapeDtypeStruct((NPOINT, B), jnp.float32),
        jax.ShapeDtypeStruct((NPOINT, B), jnp.float32),
        jax.ShapeDtypeStruct((NPOINT, B), jnp.float32),
    )
    cent_t, nx, ny, nz = pl.pallas_call(_fps_body, out_shape=out_shapes)(
        xt[0], xt[1], xt[2])
    centroids = cent_t.T
    new_xyz = jnp.stack([nx.T, ny.T, nz.T], axis=-1)
    return centroids, new_xyz


_NW = 32
_HSLICE = (B * NPOINT * K) // _NW


def _hist_body(idx_hbm, out_hbm, idx_v, tab_v):
    wid = lax.axis_index("s") * 2 + lax.axis_index("c")
    base = wid * _HSLICE
    pltpu.sync_copy(idx_hbm.at[pl.ds(base, _HSLICE)], idx_v)
    zeros16 = jnp.zeros((16,), jnp.float32)
    ones16 = jnp.ones((16,), jnp.float32)

    def zbody(i, _):
        tab_v[pl.ds(i * 16, 16)] = zeros16
        return 0

    lax.fori_loop(0, N // 16, zbody, 0)

    def body(i, _):
        v = idx_v[pl.ds(i * 16, 16)]
        plsc.addupdate_scatter(tab_v, [v], ones16)
        return 0

    lax.fori_loop(0, _HSLICE // 16, body, 0)
    pltpu.sync_copy(tab_v, out_hbm.at[wid])


def _hist_sc(idx_flat):
    mesh = plsc.VectorSubcoreMesh(core_axis_name="c", subcore_axis_name="s",
                                  num_cores=2, num_subcores=16)
    fn = pl.kernel(
        _hist_body,
        out_type=jax.ShapeDtypeStruct((_NW, N), jnp.float32),
        mesh=mesh,
        scratch_types=[
            pltpu.VMEM((_HSLICE,), jnp.int32),
            pltpu.VMEM((N,), jnp.float32),
        ],
        compiler_params=pltpu.CompilerParams(needs_layout_passes=False),
    )
    part = fn(idx_flat)
    return part.reshape(B, 2, N)


_NBIG = N
_FINF = 3.4e38


def _topk_body(p_ref, q_ref, xx_ref, qq_ref, idx_ref, d_ref):
    mm = jax.lax.dot_general(p_ref[0], q_ref[0], (((1,), (0,)), ((), ())),
                             preferred_element_type=jnp.float32)
    d_ref[...] = (-2.0 * mm + qq_ref[0]) + xx_ref[0]
    iota_f = jax.lax.broadcasted_iota(jnp.int32, (N, NPOINT), 0).astype(
        jnp.float32)

    def step(j, _):
        d = d_ref[...]
        m = jnp.min(d, axis=0, keepdims=True)
        cand = jnp.where(d == m, iota_f, _FINF)
        sel = jnp.min(cand, axis=0, keepdims=True)
        idx_ref[0, pl.ds(j, 1), :] = sel.astype(jnp.int32)
        d_ref[...] = jnp.where(iota_f == sel, _FINF, d)
        return 0

    jax.lax.fori_loop(0, K, step, 0, unroll=2)


def _topk(xyz, new_xyz_t):
    xx = jnp.sum(xyz ** 2, -1)[:, :, None]
    qq = jnp.sum(new_xyz_t.transpose(0, 2, 1) ** 2, -1)[:, None, :]
    idx_t = pl.pallas_call(
        _topk_body,
        grid=(B,),
        in_specs=[pl.BlockSpec((1, N, 3), lambda b: (b, 0, 0)),
                  pl.BlockSpec((1, 3, NPOINT), lambda b: (b, 0, 0)),
                  pl.BlockSpec((1, N, 1), lambda b: (b, 0, 0)),
                  pl.BlockSpec((1, 1, NPOINT), lambda b: (b, 0, 0))],
        out_specs=pl.BlockSpec((1, K, NPOINT), lambda b: (b, 0, 0)),
        out_shape=jax.ShapeDtypeStruct((B, K, NPOINT), jnp.int32),
        scratch_shapes=[pltpu.VMEM((N, NPOINT), jnp.float32)],
    )(xyz, new_xyz_t, xx, qq)
    return idx_t.transpose(0, 2, 1)


_GPW = (B * NPOINT) // _NW
_GCH = 4
_NCH = _GPW // _GCH
_ROWS_CH = _GCH * K


def _gmax_body(x2_hbm, idx_hbm, out_hbm, idx_v, rows_a, rows_b, outc_v,
               sem_a, sem_b):
    wid = lax.axis_index("s") * 2 + lax.axis_index("c")
    ibase = wid * _GPW * K
    pltpu.sync_copy(idx_hbm.at[pl.ds(ibase, _GPW * K)], idx_v)
    badd = jnp.full((16,), (wid // 2) * N, dtype=jnp.int32)

    def addb(i, _):
        idx_v[pl.ds(i * 16, 16)] = idx_v[pl.ds(i * 16, 16)] + badd
        return 0

    lax.fori_loop(0, (_GPW * K) // 16, addb, 0)

    def fire(c, rows_v, sem):
        pltpu.async_copy(
            x2_hbm.at[idx_v.at[pl.ds(c * _ROWS_CH, _ROWS_CH)]], rows_v, sem)

    def wait(rows_v, sem):
        pltpu.make_async_copy(
            x2_hbm.at[idx_v.at[pl.ds(0, _ROWS_CH)]], rows_v, sem).wait()

    fire(0, rows_a, sem_a)
    fire(1, rows_b, sem_b)

    def process(c, rows_v):
        for g in range(_GCH):
            accs = [rows_v[g * K, pl.ds(h * 16, 16)] for h in range(16)]

            def rbody(r, accs):
                return tuple(
                    jnp.maximum(a, rows_v[g * K + r, pl.ds(h * 16, 16)])
                    for h, a in enumerate(accs))

            accs = lax.fori_loop(1, K, rbody, tuple(accs))
            for h in range(16):
                outc_v[g, pl.ds(h * 16, 16)] = accs[h]
        pltpu.sync_copy(outc_v,
                        out_hbm.at[pl.ds(wid * _GPW + c * _GCH, _GCH)])

    def pair(p, _):
        c0 = 2 * p
        wait(rows_a, sem_a)
        process(c0, rows_a)

        @pl.when(c0 + 2 < _NCH)
        def _():
            fire(c0 + 2, rows_a, sem_a)

        wait(rows_b, sem_b)
        process(c0 + 1, rows_b)

        @pl.when(c0 + 3 < _NCH)
        def _():
            fire(c0 + 3, rows_b, sem_b)

        return 0

    lax.fori_loop(0, _NCH // 2, pair, 0)


def _gmax_sc(x2_flat, idx_flat):
    mesh = plsc.VectorSubcoreMesh(core_axis_name="c", subcore_axis_name="s",
                                  num_cores=2, num_subcores=16)
    fn = pl.kernel(
        _gmax_body,
        out_type=jax.ShapeDtypeStruct((B * NPOINT, C), jnp.float32),
        mesh=mesh,
        scratch_types=[
            pltpu.VMEM((_GPW * K,), jnp.int32),
            pltpu.VMEM((_ROWS_CH, C), jnp.float32),
            pltpu.VMEM((_ROWS_CH, C), jnp.float32),
            pltpu.VMEM((_GCH, C), jnp.float32),
            pltpu.SemaphoreType.DMA,
            pltpu.SemaphoreType.DMA,
        ],
        compiler_params=pltpu.CompilerParams(needs_layout_passes=False),
    )
    return fn(x2_flat, idx_flat)


def _acc_update(acc_ref, cnt_ref, y):
    cnt = cnt_ref[0]
    c1 = cnt[0:1] + cnt[1:2]
    ws = jnp.dot(c1, y, preferred_element_type=jnp.float32)
    wsq = jnp.dot(c1, y * y, preferred_element_type=jnp.float32)

    @pl.when(pl.program_id(0) == 0)
    def _():
        acc_ref[...] = jnp.zeros_like(acc_ref)

    acc_ref[...] += jnp.concatenate([ws, wsq], axis=0)


def _s1_body(f_ref, cnt_ref, w_ref, b_ref, y_ref, acc_ref):
    y = jnp.dot(f_ref[0], w_ref[...], preferred_element_type=jnp.float32)
    y = y + b_ref[...]
    y_ref[0] = y
    _acc_update(acc_ref, cnt_ref, y)


def _smid_body(yp_ref, cnt_ref, st_ref, g_ref, bt_ref, w_ref, b_ref,
               y_ref, acc_ref):
    scale, off = _bn_coeffs(st_ref[...], g_ref[...], bt_ref[...])
    x = _lrelu(yp_ref[0] * scale + off)
    y = jnp.dot(x, w_ref[...], preferred_element_type=jnp.float32)
    y = y + b_ref[...]
    y_ref[0] = y
    _acc_update(acc_ref, cnt_ref, y)


def _s4_body(y3_ref, y1_ref, cnt_ref, st3_ref, st1_ref, g3_ref, bt3_ref,
             g1_ref, bt1_ref, w_ref, b_ref, x1_ref, y_ref, acc_ref):
    scale3, off3 = _bn_coeffs(st3_ref[...], g3_ref[...], bt3_ref[...])
    scale1, off1 = _bn_coeffs(st1_ref[...], g1_ref[...], bt1_ref[...])
    h2 = y3_ref[0] * scale3 + off3
    xt = _lrelu(y1_ref[0] * scale1 + off1)
    x1 = _lrelu(h2 + xt)
    x1_ref[0] = x1
    y = jnp.dot(x1, w_ref[...], preferred_element_type=jnp.float32)
    y = y + b_ref[...]
    y_ref[0] = y
    _acc_update(acc_ref, cnt_ref, y)


def _s6_body(y5_ref, x1_ref, st5_ref, g5_ref, bt5_ref, x2_ref):
    scale5, off5 = _bn_coeffs(st5_ref[...], g5_ref[...], bt5_ref[...])
    x2_ref[0] = _lrelu(y5_ref[0] * scale5 + off5 + x1_ref[0])


_ROWS = pl.BlockSpec((1, N, C), lambda b: (b, 0, 0))
_CNT = pl.BlockSpec((1, 2, N), lambda b: (b, 0, 0))
_MAT = pl.BlockSpec((C, C), lambda b: (0, 0))
_VEC = pl.BlockSpec((1, C), lambda b: (0, 0))
_ACC = pl.BlockSpec((2, C), lambda b: (0, 0))

_ROWS_SHAPE = jax.ShapeDtypeStruct((B, N, C), jnp.float32)
_ACC_SHAPE = jax.ShapeDtypeStruct((2, C), jnp.float32)


def _stage1(f, cnt2, wt, bvec):
    return pl.pallas_call(
        _s1_body,
        grid=(B,),
        in_specs=[_ROWS, _CNT, _MAT, _VEC],
        out_specs=(_ROWS, _ACC),
        out_shape=(_ROWS_SHAPE, _ACC_SHAPE),
    )(f, cnt2, wt, bvec)


def _stage_mid(yp, cnt2, st, g, bt, wt, bvec):
    return pl.pallas_call(
        _smid_body,
        grid=(B,),
        in_specs=[_ROWS, _CNT, _ACC, _VEC, _VEC, _MAT, _VEC],
        out_specs=(_ROWS, _ACC),
        out_shape=(_ROWS_SHAPE, _ACC_SHAPE),
    )(yp, cnt2, st, g, bt, wt, bvec)


def _stage4(y3, y1, cnt2, st3, st1, g3, bt3, g1, bt1, wt, bvec):
    return pl.pallas_call(
        _s4_body,
        grid=(B,),
        in_specs=[_ROWS, _ROWS, _CNT, _ACC, _ACC, _VEC, _VEC, _VEC, _VEC,
                  _MAT, _VEC],
        out_specs=(_ROWS, _ROWS, _ACC),
        out_shape=(_ROWS_SHAPE, _ROWS_SHAPE, _ACC_SHAPE),
    )(y3, y1, cnt2, st3, st1, g3, bt3, g1, bt1, wt, bvec)


def _stage6(y5, x1, st5, g5, bt5):
    return pl.pallas_call(
        _s6_body,
        grid=(B,),
        in_specs=[_ROWS, _ROWS, _ACC, _VEC, _VEC],
        out_specs=_ROWS,
        out_shape=_ROWS_SHAPE,
    )(y5, x1, st5, g5, bt5)


def kernel(xyz, features, params):
    centroids, new_xyz = _fps(xyz)

    new_xyz_t = new_xyz.transpose(0, 2, 1)
    idx = _topk(xyz, new_xyz_t)

    cnt2 = _hist_sc(idx.reshape(-1).astype(jnp.int32))

    p = params
    v = lambda nm: p[nm].reshape(1, C)
    wT = lambda nm: p[nm].T

    y1, a1 = _stage1(features, cnt2, wT('W_t'), v('b_t'))
    y2, a2 = _stage_mid(y1, cnt2, a1, v('g_t'), v('bt_t'),
                        wT('W_r1a'), v('b_r1a'))
    y3, a3 = _stage_mid(y2, cnt2, a2, v('g_r1a'), v('bt_r1a'),
                        wT('W_r1b'), v('b_r1b'))
    x1, y4, a4 = _stage4(y3, y1, cnt2, a3, a1, v('g_r1b'), v('bt_r1b'),
                         v('g_t'), v('bt_t'), wT('W_r2a'), v('b_r2a'))
    y5, a5 = _stage_mid(y4, cnt2, a4, v('g_r2a'), v('bt_r2a'),
                        wT('W_r2b'), v('b_r2b'))
    x2 = _stage6(y5, x1, a5, v('g_r2b'), v('bt_r2b'))

    gm = _gmax_sc(x2.reshape(B * N, C), idx.reshape(-1).astype(jnp.int32))
    sub_features = gm.reshape(B, NPOINT, C).transpose(0, 2, 1)

    return (new_xyz_t, sub_features)


# speedup vs baseline: 3.8575x; 1.0369x over previous
import functools

import jax
import jax.numpy as jnp
from jax import lax
from jax.experimental import pallas as pl
from jax.experimental.pallas import tpu as pltpu
from jax.experimental.pallas import tpu_sc as plsc

B, N, C, NPOINT, K = 16, 2048, 256, 512, 32
CNT_TOT = float(B * NPOINT * K)
EPS = 1e-5


def _lrelu(v):
    return jnp.where(v >= 0, v, 0.1 * v)


def _bn_coeffs(acc, g, bt):
    mean = acc[0:1] / CNT_TOT
    var = acc[1:2] / CNT_TOT - mean * mean
    scale = g * jax.lax.rsqrt(var + EPS)
    off = bt - mean * scale
    return scale, off


def _fps_body(x_ref, y_ref, z_ref, cent_ref, nx_ref, ny_ref, nz_ref):
    x = x_ref[...]
    y = y_ref[...]
    z = z_ref[...]
    iota = jax.lax.broadcasted_iota(jnp.int32, (B, N), 1)

    def step(i, carry):
        distance, farthest = carry
        cent_ref[pl.ds(i, 1), :] = farthest.reshape(1, B)
        mask = iota == farthest
        zero = jnp.zeros_like(x)
        cx = jnp.sum(jnp.where(mask, x, zero), axis=1, keepdims=True)
        cy = jnp.sum(jnp.where(mask, y, zero), axis=1, keepdims=True)
        cz = jnp.sum(jnp.where(mask, z, zero), axis=1, keepdims=True)
        nx_ref[pl.ds(i, 1), :] = cx.reshape(1, B)
        ny_ref[pl.ds(i, 1), :] = cy.reshape(1, B)
        nz_ref[pl.ds(i, 1), :] = cz.reshape(1, B)
        dx = x - cx
        dy = y - cy
        dz = z - cz
        dist = dx * dx + dy * dy + dz * dz
        distance = jnp.minimum(distance, dist)
        m = jnp.max(distance, axis=1, keepdims=True)
        far = jnp.min(jnp.where(distance == m, iota, N), axis=1, keepdims=True)
        return distance, far

    init = (jnp.full((B, N), 1e10, dtype=jnp.float32),
            jnp.zeros((B, 1), dtype=jnp.int32))
    jax.lax.fori_loop(0, NPOINT, step, init)


def _fps(xyz):
    xt = xyz.transpose(2, 0, 1)
    out_shapes = (
        jax.ShapeDtypeStruct((NPOINT, B), jnp.int32),
        jax.ShapeDtypeStruct((NPOINT, B), jnp.float32),
        jax.ShapeDtypeStruct((NPOINT, B), jnp.float32),
        jax.ShapeDtypeStruct((NPOINT, B), jnp.float32),
    )
    cent_t, nx, ny, nz = pl.pallas_call(_fps_body, out_shape=out_shapes)(
        xt[0], xt[1], xt[2])
    centroids = cent_t.T
    new_xyz = jnp.stack([nx.T, ny.T, nz.T], axis=-1)
    return centroids, new_xyz


_NW = 32
_HSLICE = (B * NPOINT * K) // _NW


def _hist_body(idx_hbm, out_hbm, idx_v, tab_v):
    wid = lax.axis_index("s") * 2 + lax.axis_index("c")
    base = wid * _HSLICE
    pltpu.sync_copy(idx_hbm.at[pl.ds(base, _HSLICE)], idx_v)
    zeros16 = jnp.zeros((16,), jnp.float32)
    ones16 = jnp.ones((16,), jnp.float32)

    def zbody(i, _):
        tab_v[pl.ds(i * 16, 16)] = zeros16
        return 0

    lax.fori_loop(0, N // 16, zbody, 0)

    def body(i, _):
        v = idx_v[pl.ds(i * 16, 16)]
        plsc.addupdate_scatter(tab_v, [v], ones16)
        return 0

    lax.fori_loop(0, _HSLICE // 16, body, 0)
    pltpu.sync_copy(tab_v, out_hbm.at[wid])


def _hist_sc(idx_flat):
    mesh = plsc.VectorSubcoreMesh(core_axis_name="c", subcore_axis_name="s",
                                  num_cores=2, num_subcores=16)
    fn = pl.kernel(
        _hist_body,
        out_type=jax.ShapeDtypeStruct((_NW, N), jnp.float32),
        mesh=mesh,
        scratch_types=[
            pltpu.VMEM((_HSLICE,), jnp.int32),
            pltpu.VMEM((N,), jnp.float32),
        ],
        compiler_params=pltpu.CompilerParams(needs_layout_passes=False),
    )
    part = fn(idx_flat)
    return part.reshape(B, 2, N)


_NBIG = N
_FINF = 3.4e38


def _topk_body(p_ref, q_ref, xx_ref, qq_ref, idx_ref, d_ref):
    mm = jax.lax.dot_general(p_ref[0], q_ref[0], (((1,), (0,)), ((), ())),
                             preferred_element_type=jnp.float32)
    d_ref[...] = (-2.0 * mm + qq_ref[0]) + xx_ref[0]
    iota_f = jax.lax.broadcasted_iota(jnp.int32, (N, NPOINT), 0).astype(
        jnp.float32)

    def step(j, _):
        d = d_ref[...]
        m = jnp.min(d, axis=0, keepdims=True)
        cand = jnp.where(d == m, iota_f, _FINF)
        sel = jnp.min(cand, axis=0, keepdims=True)
        idx_ref[0, pl.ds(j, 1), :] = sel.astype(jnp.int32)
        d_ref[...] = jnp.where(iota_f == sel, _FINF, d)
        return 0

    jax.lax.fori_loop(0, K, step, 0, unroll=4)


def _topk(xyz, new_xyz_t):
    xx = jnp.sum(xyz ** 2, -1)[:, :, None]
    qq = jnp.sum(new_xyz_t.transpose(0, 2, 1) ** 2, -1)[:, None, :]
    idx_t = pl.pallas_call(
        _topk_body,
        grid=(B,),
        in_specs=[pl.BlockSpec((1, N, 3), lambda b: (b, 0, 0)),
                  pl.BlockSpec((1, 3, NPOINT), lambda b: (b, 0, 0)),
                  pl.BlockSpec((1, N, 1), lambda b: (b, 0, 0)),
                  pl.BlockSpec((1, 1, NPOINT), lambda b: (b, 0, 0))],
        out_specs=pl.BlockSpec((1, K, NPOINT), lambda b: (b, 0, 0)),
        out_shape=jax.ShapeDtypeStruct((B, K, NPOINT), jnp.int32),
        scratch_shapes=[pltpu.VMEM((N, NPOINT), jnp.float32)],
    )(xyz, new_xyz_t, xx, qq)
    return idx_t.transpose(0, 2, 1)


_GPW = (B * NPOINT) // _NW
_GCH = 4
_NCH = _GPW // _GCH
_ROWS_CH = _GCH * K


def _gmax_body(x2_hbm, idx_hbm, out_hbm, idx_v, rows_a, rows_b, outc_v,
               sem_a, sem_b):
    wid = lax.axis_index("s") * 2 + lax.axis_index("c")
    ibase = wid * _GPW * K
    pltpu.sync_copy(idx_hbm.at[pl.ds(ibase, _GPW * K)], idx_v)
    badd = jnp.full((16,), (wid // 2) * N, dtype=jnp.int32)

    def addb(i, _):
        idx_v[pl.ds(i * 16, 16)] = idx_v[pl.ds(i * 16, 16)] + badd
        return 0

    lax.fori_loop(0, (_GPW * K) // 16, addb, 0)

    def fire(c, rows_v, sem):
        pltpu.async_copy(
            x2_hbm.at[idx_v.at[pl.ds(c * _ROWS_CH, _ROWS_CH)]], rows_v, sem)

    def wait(rows_v, sem):
        pltpu.make_async_copy(
            x2_hbm.at[idx_v.at[pl.ds(0, _ROWS_CH)]], rows_v, sem).wait()

    fire(0, rows_a, sem_a)
    fire(1, rows_b, sem_b)

    def process(c, rows_v):
        for g in range(_GCH):
            accs = [rows_v[g * K, pl.ds(h * 16, 16)] for h in range(16)]

            def rbody(r, accs):
                return tuple(
                    jnp.maximum(a, rows_v[g * K + r, pl.ds(h * 16, 16)])
                    for h, a in enumerate(accs))

            accs = lax.fori_loop(1, K, rbody, tuple(accs))
            for h in range(16):
                outc_v[g, pl.ds(h * 16, 16)] = accs[h]
        pltpu.sync_copy(outc_v,
                        out_hbm.at[pl.ds(wid * _GPW + c * _GCH, _GCH)])

    def pair(p, _):
        c0 = 2 * p
        wait(rows_a, sem_a)
        process(c0, rows_a)

        @pl.when(c0 + 2 < _NCH)
        def _():
            fire(c0 + 2, rows_a, sem_a)

        wait(rows_b, sem_b)
        process(c0 + 1, rows_b)

        @pl.when(c0 + 3 < _NCH)
        def _():
            fire(c0 + 3, rows_b, sem_b)

        return 0

    lax.fori_loop(0, _NCH // 2, pair, 0)


def _gmax_sc(x2_flat, idx_flat):
    mesh = plsc.VectorSubcoreMesh(core_axis_name="c", subcore_axis_name="s",
                                  num_cores=2, num_subcores=16)
    fn = pl.kernel(
        _gmax_body,
        out_type=jax.ShapeDtypeStruct((B * NPOINT, C), jnp.float32),
        mesh=mesh,
        scratch_types=[
            pltpu.VMEM((_GPW * K,), jnp.int32),
            pltpu.VMEM((_ROWS_CH, C), jnp.float32),
            pltpu.VMEM((_ROWS_CH, C), jnp.float32),
            pltpu.VMEM((_GCH, C), jnp.float32),
            pltpu.SemaphoreType.DMA,
            pltpu.SemaphoreType.DMA,
        ],
        compiler_params=pltpu.CompilerParams(needs_layout_passes=False),
    )
    return fn(x2_flat, idx_flat)


def _acc_update(acc_ref, cnt_ref, y):
    cnt = cnt_ref[0]
    c1 = cnt[0:1] + cnt[1:2]
    ws = jnp.dot(c1, y, preferred_element_type=jnp.float32)
    wsq = jnp.dot(c1, y * y, preferred_element_type=jnp.float32)

    @pl.when(pl.program_id(0) == 0)
    def _():
        acc_ref[...] = jnp.zeros_like(acc_ref)

    acc_ref[...] += jnp.concatenate([ws, wsq], axis=0)


def _s1_body(f_ref, cnt_ref, w_ref, b_ref, y_ref, acc_ref):
    y = jnp.dot(f_ref[0], w_ref[...], preferred_element_type=jnp.float32)
    y = y + b_ref[...]
    y_ref[0] = y
    _acc_update(acc_ref, cnt_ref, y)


def _smid_body(yp_ref, cnt_ref, st_ref, g_ref, bt_ref, w_ref, b_ref,
               y_ref, acc_ref):
    scale, off = _bn_coeffs(st_ref[...], g_ref[...], bt_ref[...])
    x = _lrelu(yp_ref[0] * scale + off)
    y = jnp.dot(x, w_ref[...], preferred_element_type=jnp.float32)
    y = y + b_ref[...]
    y_ref[0] = y
    _acc_update(acc_ref, cnt_ref, y)


def _s4_body(y3_ref, y1_ref, cnt_ref, st3_ref, st1_ref, g3_ref, bt3_ref,
             g1_ref, bt1_ref, w_ref, b_ref, x1_ref, y_ref, acc_ref):
    scale3, off3 = _bn_coeffs(st3_ref[...], g3_ref[...], bt3_ref[...])
    scale1, off1 = _bn_coeffs(st1_ref[...], g1_ref[...], bt1_ref[...])
    h2 = y3_ref[0] * scale3 + off3
    xt = _lrelu(y1_ref[0] * scale1 + off1)
    x1 = _lrelu(h2 + xt)
    x1_ref[0] = x1
    y = jnp.dot(x1, w_ref[...], preferred_element_type=jnp.float32)
    y = y + b_ref[...]
    y_ref[0] = y
    _acc_update(acc_ref, cnt_ref, y)


def _s6_body(y5_ref, x1_ref, st5_ref, g5_ref, bt5_ref, x2_ref):
    scale5, off5 = _bn_coeffs(st5_ref[...], g5_ref[...], bt5_ref[...])
    x2_ref[0] = _lrelu(y5_ref[0] * scale5 + off5 + x1_ref[0])


_ROWS = pl.BlockSpec((1, N, C), lambda b: (b, 0, 0))
_CNT = pl.BlockSpec((1, 2, N), lambda b: (b, 0, 0))
_MAT = pl.BlockSpec((C, C), lambda b: (0, 0))
_VEC = pl.BlockSpec((1, C), lambda b: (0, 0))
_ACC = pl.BlockSpec((2, C), lambda b: (0, 0))

_ROWS_SHAPE = jax.ShapeDtypeStruct((B, N, C), jnp.float32)
_ACC_SHAPE = jax.ShapeDtypeStruct((2, C), jnp.float32)


def _stage1(f, cnt2, wt, bvec):
    return pl.pallas_call(
        _s1_body,
        grid=(B,),
        in_specs=[_ROWS, _CNT, _MAT, _VEC],
        out_specs=(_ROWS, _ACC),
        out_shape=(_ROWS_SHAPE, _ACC_SHAPE),
    )(f, cnt2, wt, bvec)


def _stage_mid(yp, cnt2, st, g, bt, wt, bvec):
    return pl.pallas_call(
        _smid_body,
        grid=(B,),
        in_specs=[_ROWS, _CNT, _ACC, _VEC, _VEC, _MAT, _VEC],
        out_specs=(_ROWS, _ACC),
        out_shape=(_ROWS_SHAPE, _ACC_SHAPE),
    )(yp, cnt2, st, g, bt, wt, bvec)


def _stage4(y3, y1, cnt2, st3, st1, g3, bt3, g1, bt1, wt, bvec):
    return pl.pallas_call(
        _s4_body,
        grid=(B,),
        in_specs=[_ROWS, _ROWS, _CNT, _ACC, _ACC, _VEC, _VEC, _VEC, _VEC,
                  _MAT, _VEC],
        out_specs=(_ROWS, _ROWS, _ACC),
        out_shape=(_ROWS_SHAPE, _ROWS_SHAPE, _ACC_SHAPE),
    )(y3, y1, cnt2, st3, st1, g3, bt3, g1, bt1, wt, bvec)


def _stage6(y5, x1, st5, g5, bt5):
    return pl.pallas_call(
        _s6_body,
        grid=(B,),
        in_specs=[_ROWS, _ROWS, _ACC, _VEC, _VEC],
        out_specs=_ROWS,
        out_shape=_ROWS_SHAPE,
    )(y5, x1, st5, g5, bt5)


def kernel(xyz, features, params):
    centroids, new_xyz = _fps(xyz)

    new_xyz_t = new_xyz.transpose(0, 2, 1)
    idx = _topk(xyz, new_xyz_t)

    cnt2 = _hist_sc(idx.reshape(-1).astype(jnp.int32))

    p = params
    v = lambda nm: p[nm].reshape(1, C)
    wT = lambda nm: p[nm].T

    y1, a1 = _stage1(features, cnt2, wT('W_t'), v('b_t'))
    y2, a2 = _stage_mid(y1, cnt2, a1, v('g_t'), v('bt_t'),
                        wT('W_r1a'), v('b_r1a'))
    y3, a3 = _stage_mid(y2, cnt2, a2, v('g_r1a'), v('bt_r1a'),
                        wT('W_r1b'), v('b_r1b'))
    x1, y4, a4 = _stage4(y3, y1, cnt2, a3, a1, v('g_r1b'), v('bt_r1b'),
                         v('g_t'), v('bt_t'), wT('W_r2a'), v('b_r2a'))
    y5, a5 = _stage_mid(y4, cnt2, a4, v('g_r2a'), v('bt_r2a'),
                        wT('W_r2b'), v('b_r2b'))
    x2 = _stage6(y5, x1, a5, v('g_r2b'), v('bt_r2b'))

    gm = _gmax_sc(x2.reshape(B * N, C), idx.reshape(-1).astype(jnp.int32))
    sub_features = gm.reshape(B, NPOINT, C).transpose(0, 2, 1)

    return (new_xyz_t, sub_features)
